# baseline (speedup 1.0000x reference)
import functools

import jax
import jax.numpy as jnp
from jax import lax
from jax.experimental import pallas as pl
from jax.experimental.pallas import tpu as pltpu
from jax.experimental.pallas import tpu_sc as plsc

NUM_CORES = 2
NUM_SUBCORES = 16
NUM_WORKERS = NUM_CORES * NUM_SUBCORES
CHUNK = 1600
DETILE_W = 8192
DETILE_SLAB = DETILE_W // 4


def _detile_table(table_t, v, d):
    w = DETILE_W
    nblk = pl.cdiv(v, w)
    slab = w // 4

    def body(in_ref, out_ref):
        xtp = jax.lax.dot_general(
            in_ref[...], jnp.eye(d, 128, dtype=jnp.float32),
            (((0,), (0,)), ((), ())),
            preferred_element_type=jnp.float32)
        out = xtp[0:slab, :]
        for a in range(1, 4):
            out = out + jnp.roll(xtp[a * slab:(a + 1) * slab, :], a * d, 1)
        out_ref[...] = out

    return pl.pallas_call(
        body,
        grid=(nblk,),
        in_specs=[pl.BlockSpec((d, w), lambda i: (0, i))],
        out_specs=pl.BlockSpec((slab, 128), lambda i: (i, 0)),
        out_shape=jax.ShapeDtypeStruct((nblk * slab, 128), jnp.float32),
    )(table_t)


SBLK = 4


@functools.partial(jax.jit, static_argnums=(2,))
def _gather_rows(idx, table, out_shape):
    bsz, seq, d = out_shape
    v = table.shape[0]
    bpw = bsz // NUM_WORKERS
    idx_per_w = bpw * seq
    n_blocks = seq // SBLK
    gn = SBLK * bpw

    tflat = _detile_table(table.T, v, d)
    t2 = tflat.reshape(-1, d)

    mesh = plsc.VectorSubcoreMesh(core_axis_name="c", subcore_axis_name="s")

    @functools.partial(
        pl.kernel,
        mesh=mesh,
        out_type=jax.ShapeDtypeStruct((seq, d, bsz), jnp.float32),
        scratch_types=[
            pltpu.VMEM((idx_per_w + 16,), jnp.int32),
            pltpu.VMEM((idx_per_w,), jnp.int32),
            pltpu.VMEM((gn, d), jnp.float32),
            pltpu.VMEM((gn, d), jnp.float32),
            pltpu.VMEM((SBLK, d, bpw), jnp.float32),
            pltpu.VMEM((SBLK, d, bpw), jnp.float32),
            pltpu.SemaphoreType.DMA((2,)),
            pltpu.SemaphoreType.DMA((2,)),
        ],
        compiler_params=pltpu.CompilerParams(
            use_tc_tiling_on_sc=False, needs_layout_passes=False),
    )
    def k(table_hbm, idx_hbm, out_hbm, idx_raw, idx_t,
          rows0, rows1, stage0, stage1, sem_g, sem_o):
        wid = lax.axis_index("s") * NUM_CORES + lax.axis_index("c")
        b0 = wid * bpw
        iota16 = lax.iota(jnp.int32, 16)
        iota_bpw = iota16 * bpw
        cvecs = [jnp.full((16,), c, jnp.int32) for c in range(d)]

        pltpu.sync_copy(idx_hbm.at[pl.ds(b0 * seq, idx_per_w)],
                        idx_raw.at[pl.ds(0, idx_per_w)])

        n_sg = seq // 16
        tail = seq - n_sg * 16

        iota_seq = iota16 * seq

        def ts(s, carry):
            def tg(kg, carry2):
                vec = plsc.load_gather(
                    idx_raw, [iota_seq + (kg * 16 * seq + s)])
                idx_t[pl.ds(s * bpw + kg * 16, 16)] = vec
                return carry2

            lax.fori_loop(0, bpw // 16, tg, 0)
            return carry

        lax.fori_loop(0, seq, ts, 0)

        def start_g(i, rows, slot):
            pltpu.make_async_copy(
                table_hbm.at[idx_t.at[pl.ds(i * gn, gn)]],
                rows, sem_g.at[slot]).start()

        def wait_g(rows, slot):
            pltpu.make_async_copy(
                table_hbm.at[idx_t.at[pl.ds(0, gn)]],
                rows, sem_g.at[slot]).wait()

        def out_slice(i):
            return out_hbm.at[pl.ds(i * SBLK, SBLK), :, pl.ds(b0, bpw)]

        def start_out(i, stage, slot):
            pltpu.make_async_copy(stage, out_slice(i), sem_o.at[slot]).start()

        def wait_out(i, stage, slot):
            pltpu.make_async_copy(stage, out_slice(i), sem_o.at[slot]).wait()

        def trans(rows, stage):
            def g16(g, carry):
                js = g >> 3
                kb0 = (g & 7) * 16
                rowv = iota16 + g * 16
                for c in range(d):
                    vec = plsc.load_gather(rows, [rowv, cvecs[c]])
                    stage[js, c, pl.ds(kb0, 16)] = vec
                return carry

            lax.fori_loop(0, gn // 16, g16, 0)

        def half_step(i, rows, stage, slot):
            wait_g(rows, slot)

            @pl.when(i >= 2)
            def _():
                wait_out(i - 2, stage, slot)

            trans(rows, stage)
            start_out(i, stage, slot)

            @pl.when(i + 2 < n_blocks)
            def _():
                start_g(i + 2, rows, slot)

        start_g(0, rows0, 0)
        start_g(1, rows1, 1)

        def body(i2, carry):
            half_step(2 * i2, rows0, stage0, 0)
            half_step(2 * i2 + 1, rows1, stage1, 1)
            return carry

        lax.fori_loop(0, n_blocks // 2, body, 0)

        wait_out(n_blocks - 2, stage0, 0)
        wait_out(n_blocks - 1, stage1, 1)

    return k(t2, idx)


def kernel(token_ids, table):
    bsz, seq = token_ids.shape
    v, d = table.shape
    flat = token_ids.reshape(-1).astype(jnp.int32)
    w, slab = DETILE_W, DETILE_SLAB
    shift = slab.bit_length() - 1
    midx = ((flat & ~(w - 1)) + ((flat & (slab - 1)) << 2)
            + ((flat & (w - 1)) >> shift))
    out_t = _gather_rows(midx, table, (bsz, seq, d))
    return out_t.transpose(2, 0, 1)


# speedup vs baseline: 1.3223x; 1.3223x over previous
import functools

import jax
import jax.numpy as jnp
from jax import lax
from jax.experimental import pallas as pl
from jax.experimental.pallas import tpu as pltpu
from jax.experimental.pallas import tpu_sc as plsc

NUM_CORES = 2
NUM_SUBCORES = 16
NUM_WORKERS = NUM_CORES * NUM_SUBCORES
CHUNK = 1600
DETILE_W = 16384
DETILE_SLAB = DETILE_W // 4


def _detile_table(table_t, v, d):
    w = DETILE_W
    nblk = pl.cdiv(v, w)
    slab = w // 4

    def body(in_ref, out_ref):
        xtp = jax.lax.dot_general(
            in_ref[...], jnp.eye(d, 128, dtype=jnp.float32),
            (((0,), (0,)), ((), ())),
            preferred_element_type=jnp.float32)
        out = xtp[0:slab, :]
        for a in range(1, 4):
            out = out + jnp.roll(xtp[a * slab:(a + 1) * slab, :], a * d, 1)
        out_ref[...] = out

    return pl.pallas_call(
        body,
        grid=(nblk,),
        in_specs=[pl.BlockSpec((d, w), lambda i: (0, i))],
        out_specs=pl.BlockSpec((slab, 128), lambda i: (i, 0)),
        out_shape=jax.ShapeDtypeStruct((nblk * slab, 128), jnp.float32),
    )(table_t)


@functools.partial(jax.jit, static_argnums=(2,))
def _gather_rows(idx, table, out_shape):
    n = idx.shape[0]
    bsz, seq, d = out_shape
    v = table.shape[0]
    rows_per_chunk = CHUNK // seq
    b_per_w = n // NUM_WORKERS
    n_chunks = b_per_w // CHUNK

    tflat = _detile_table(table.T, v, d)
    t2 = tflat.reshape(-1, d)

    mesh = plsc.VectorSubcoreMesh(core_axis_name="c", subcore_axis_name="s")

    @functools.partial(
        pl.kernel,
        mesh=mesh,
        out_type=jax.ShapeDtypeStruct((bsz, seq, d), jnp.float32),
        scratch_types=[
            pltpu.VMEM((2, CHUNK), jnp.int32),
            pltpu.VMEM((2, CHUNK, d), jnp.float32),
            pltpu.SemaphoreType.DMA((2,)),
            pltpu.SemaphoreType.DMA((2,)),
        ],
        compiler_params=pltpu.CompilerParams(use_tc_tiling_on_sc=False),
    )
    def k(table_hbm, idx_hbm, out_hbm, idx_v, rows_v, sem_g, sem_o):
        wid = lax.axis_index("s") * NUM_CORES + lax.axis_index("c")
        base = wid * b_per_w
        row_base = wid * (b_per_w // seq)

        def start_gather(c, b):
            pltpu.sync_copy(idx_hbm.at[pl.ds(base + c * CHUNK, CHUNK)],
                            idx_v.at[b])
            pltpu.make_async_copy(
                table_hbm.at[idx_v.at[b]], rows_v.at[b], sem_g.at[b]).start()

        def wait_gather(b):
            pltpu.make_async_copy(
                table_hbm.at[idx_v.at[b]], rows_v.at[b], sem_g.at[b]).wait()

        def start_out(c, b):
            for j in range(rows_per_chunk):
                pltpu.make_async_copy(
                    rows_v.at[b, pl.ds(j * seq, seq)],
                    out_hbm.at[row_base + c * rows_per_chunk + j],
                    sem_o.at[b]).start()

        def wait_out(c, b):
            for j in range(rows_per_chunk):
                pltpu.make_async_copy(
                    rows_v.at[b, pl.ds(j * seq, seq)],
                    out_hbm.at[row_base + c * rows_per_chunk + j],
                    sem_o.at[b]).wait()

        start_gather(0, 0)

        def body(c, carry):
            b = lax.rem(c, 2)
            pb = lax.rem(c - 1, 2)

            @pl.when(c >= 2)
            def _():
                wait_out(c - 2, b)

            start_gather(c, b)
            wait_gather(pb)
            start_out(c - 1, pb)
            return carry

        lax.fori_loop(1, n_chunks, body, 0)

        bl = (n_chunks - 1) % 2
        wait_gather(bl)
        start_out(n_chunks - 1, bl)
        wait_out(n_chunks - 2, 1 - bl)
        wait_out(n_chunks - 1, bl)

    return k(t2, idx)


def kernel(token_ids, table):
    bsz, seq = token_ids.shape
    v, d = table.shape
    flat = token_ids.reshape(-1).astype(jnp.int32)
    w, slab = DETILE_W, DETILE_SLAB
    shift = slab.bit_length() - 1
    midx = ((flat & ~(w - 1)) + ((flat & (slab - 1)) << 2)
            + ((flat & (w - 1)) >> shift))
    return _gather_rows(midx, table, (bsz, seq, d))
